# Initial kernel scaffold; baseline (speedup 1.0000x reference)
import jax
import jax.numpy as jnp
from jax.experimental import pallas as pl


def kernel(x, edge_index, neg_edge_index, W1, b1, W2, b2, lpW1, lpb1, lpW2, lpb2, lpW3, lpb3):
    raise NotImplementedError("write your pallas kernel here")


# speedup vs baseline: 4.3128x; 4.3128x over previous
import functools

import jax
import jax.numpy as jnp
from jax import lax
from jax.experimental import pallas as pl
from jax.experimental.pallas import tpu as pltpu
from jax.experimental.pallas import tpu_sc as plsc

N = 10000
E = 320000
D_IN = 128
HID = 64

NC = 2
NS = 16
NW = NC * NS

CH = 80
EPT = E // NW
STEPS = EPT // CH
NPAD = 10240
RPT = NPAD // NS

DEGW = 16

_mesh = plsc.VectorSubcoreMesh(core_axis_name="c", subcore_axis_name="s")


def _wid():
    return lax.axis_index("c") * NS + lax.axis_index("s")


def _deg_body(dst_hbm, ones_hbm, zeros_hbm, out_hbm, idx_v, ones_v, acc_sp):
    sub = lax.axis_index("s")
    core = lax.axis_index("c")
    wid = core * NS + sub
    base = pl.multiple_of(wid * EPT, CH)

    r0 = pl.multiple_of(sub * RPT, RPT)
    pltpu.sync_copy(zeros_hbm.at[pl.ds(r0, RPT)], acc_sp.at[pl.ds(r0, RPT)])
    pltpu.sync_copy(ones_hbm, ones_v)
    plsc.subcore_barrier()

    def step(i, _):
        off = pl.multiple_of(base + i * CH, CH)
        pltpu.sync_copy(dst_hbm.at[pl.ds(off, CH)], idx_v)
        pltpu.sync_copy(ones_v, acc_sp.at[idx_v], add=True)
        return _

    lax.fori_loop(0, STEPS, step, None)
    plsc.subcore_barrier()
    pltpu.sync_copy(acc_sp.at[pl.ds(r0, RPT)],
                    out_hbm.at[pl.ds(core * NPAD + r0, RPT)])


_deg_kernel = pl.kernel(
    _deg_body,
    out_type=jax.ShapeDtypeStruct((NC * NPAD, DEGW), jnp.float32),
    mesh=_mesh,
    compiler_params=pltpu.CompilerParams(use_tc_tiling_on_sc=False),
    scratch_types=[
        pltpu.VMEM((CH,), jnp.int32),
        pltpu.VMEM((CH, DEGW), jnp.float32),
        pltpu.VMEM_SHARED((NPAD, DEGW), jnp.float32),
    ],
)


def _agg_body(hws_hbm, src_hbm, dst_hbm, zeros_hbm, out_hbm,
              sidx_v, didx_v, rows_v, acc_sp, sem):
    sub = lax.axis_index("s")
    core = lax.axis_index("c")
    wid = core * NS + sub
    base = pl.multiple_of(wid * EPT, CH)

    r0 = pl.multiple_of(sub * RPT, RPT)
    pltpu.sync_copy(zeros_hbm.at[pl.ds(r0, RPT)], acc_sp.at[pl.ds(r0, RPT)])
    plsc.subcore_barrier()

    def step(i, _):
        off = pl.multiple_of(base + i * CH, CH)
        pltpu.sync_copy(src_hbm.at[pl.ds(off, CH)], sidx_v)
        pltpu.sync_copy(dst_hbm.at[pl.ds(off, CH)], didx_v)
        pltpu.async_copy(hws_hbm.at[sidx_v], rows_v, sem).wait()
        pltpu.sync_copy(rows_v, acc_sp.at[didx_v], add=True)
        return _

    lax.fori_loop(0, STEPS, step, None)
    plsc.subcore_barrier()
    pltpu.sync_copy(acc_sp.at[pl.ds(r0, RPT)],
                    out_hbm.at[pl.ds(core * NPAD + r0, RPT)])


_agg_kernel = pl.kernel(
    _agg_body,
    out_type=jax.ShapeDtypeStruct((NC * NPAD, HID), jnp.float32),
    mesh=_mesh,
    compiler_params=pltpu.CompilerParams(use_tc_tiling_on_sc=False),
    scratch_types=[
        pltpu.VMEM((CH,), jnp.int32),
        pltpu.VMEM((CH,), jnp.int32),
        pltpu.VMEM((CH, HID), jnp.float32),
        pltpu.VMEM_SHARED((NPAD, HID), jnp.float32),
        pltpu.SemaphoreType.DMA,
    ],
)


NJOBS = 4


def _lpgather_body(z_hbm, idx_hbm, out_hbm, idx_v, rows_v, sem):
    wid = _wid()
    tbase = pl.multiple_of(wid * EPT, CH)

    def step(t, _):
        j = t // STEPS
        i = t - j * STEPS
        off = pl.multiple_of(j * E + tbase + i * CH, CH)
        pltpu.sync_copy(idx_hbm.at[pl.ds(off, CH)], idx_v)
        pltpu.async_copy(z_hbm.at[idx_v], rows_v, sem).wait()
        pltpu.sync_copy(rows_v, out_hbm.at[pl.ds(off, CH)])
        return _

    lax.fori_loop(0, NJOBS * STEPS, step, None)


_lpgather_kernel = pl.kernel(
    _lpgather_body,
    out_type=jax.ShapeDtypeStruct((NJOBS * E, HID), jnp.float32),
    mesh=_mesh,
    compiler_params=pltpu.CompilerParams(use_tc_tiling_on_sc=False),
    scratch_types=[
        pltpu.VMEM((CH,), jnp.int32),
        pltpu.VMEM((CH, HID), jnp.float32),
        pltpu.SemaphoreType.DMA,
    ],
)


def _prescale_body(degp_ref, x_ref, w_ref, hws_ref, dinv_ref):
    deg = degp_ref[0, 0:N, 0:1] + degp_ref[1, 0:N, 0:1] + 1.0
    dinv = lax.rsqrt(deg)
    hw = jnp.dot(x_ref[...], w_ref[...], preferred_element_type=jnp.float32)
    hws_ref[...] = hw * dinv
    dinv_ref[...] = dinv


def _prescale(degp, x, w):
    return pl.pallas_call(
        _prescale_body,
        out_shape=(jax.ShapeDtypeStruct((N, HID), jnp.float32),
                   jax.ShapeDtypeStruct((N, 1), jnp.float32)),
    )(degp, x, w)


def _post_mid_body(aggp_ref, hws_ref, dinv_ref, b_ref, w_ref, out_ref):
    acc = aggp_ref[0, 0:N] + aggp_ref[1, 0:N] + hws_ref[...]
    h = jnp.maximum(acc * dinv_ref[...] + b_ref[...], 0.0)
    out_ref[...] = jnp.dot(h, w_ref[...],
                           preferred_element_type=jnp.float32) * dinv_ref[...]


def _post_mid(aggp, hws, dinv, b, w):
    return pl.pallas_call(
        _post_mid_body,
        out_shape=jax.ShapeDtypeStruct((N, HID), jnp.float32),
    )(aggp, hws, dinv, b, w)


def _post_final_body(aggp_ref, hws_ref, dinv_ref, b_ref, z_ref):
    acc = aggp_ref[0, 0:N] + aggp_ref[1, 0:N] + hws_ref[...]
    z_ref[...] = jnp.maximum(acc * dinv_ref[...] + b_ref[...], 0.0)


def _post_final(aggp, hws, dinv, b):
    return pl.pallas_call(
        _post_final_body,
        out_shape=jax.ShapeDtypeStruct((N, HID), jnp.float32),
    )(aggp, hws, dinv, b)


EBLK = 2560
NEBLK = E // EBLK


def _mlp_body(g_ref, wa_ref, wb_ref, b1_ref, w2_ref, b2_ref, w3_ref, b3_ref,
              out_ref):
    for s in range(2):
        g0 = g_ref[s, 0]
        g1 = g_ref[s, 1]
        h1 = jnp.dot(g0, wa_ref[...], preferred_element_type=jnp.float32)
        h1 = h1 + jnp.dot(g1, wb_ref[...], preferred_element_type=jnp.float32)
        h1 = jnp.maximum(h1 + b1_ref[...], 0.0)
        h2 = jnp.dot(h1, w2_ref[...], preferred_element_type=jnp.float32)
        h2 = jnp.maximum(h2 + b2_ref[...], 0.0)
        v = jnp.sum(h2 * w3_ref[...], axis=1) + b3_ref[0, 0]
        out_ref[s, :] = v


def _link_mlp(g, wa, wb, b1, w2, b2, w3, b3):
    return pl.pallas_call(
        _mlp_body,
        grid=(NEBLK,),
        in_specs=[
            pl.BlockSpec((2, 2, EBLK, HID), lambda i: (0, 0, i, 0)),
            pl.BlockSpec((HID, 2 * HID), lambda i: (0, 0)),
            pl.BlockSpec((HID, 2 * HID), lambda i: (0, 0)),
            pl.BlockSpec((1, 2 * HID), lambda i: (0, 0)),
            pl.BlockSpec((2 * HID, HID), lambda i: (0, 0)),
            pl.BlockSpec((1, HID), lambda i: (0, 0)),
            pl.BlockSpec((1, HID), lambda i: (0, 0)),
            pl.BlockSpec((1, 1), lambda i: (0, 0)),
        ],
        out_specs=pl.BlockSpec((2, EBLK), lambda i: (0, i)),
        out_shape=jax.ShapeDtypeStruct((2, E), jnp.float32),
    )(g, wa, wb, b1, w2, b2, w3, b3)


def kernel(x, edge_index, neg_edge_index, W1, b1, W2, b2,
           lpW1, lpb1, lpW2, lpb2, lpW3, lpb3):
    src = edge_index[0]
    dst = edge_index[1]
    zeros_h = jnp.zeros((NPAD, HID), jnp.float32)
    zeros_d = jnp.zeros((NPAD, DEGW), jnp.float32)
    ones_d = jnp.ones((CH, DEGW), jnp.float32)

    degp = _deg_kernel(dst, ones_d, zeros_d)
    degp = degp.reshape(NC, NPAD, DEGW)

    hws1, dinv = _prescale(degp, x, W1)
    agg1 = _agg_kernel(hws1, src, dst, zeros_h).reshape(NC, NPAD, HID)
    hws2 = _post_mid(agg1, hws1, dinv, b1.reshape(1, HID), W2)
    agg2 = _agg_kernel(hws2, src, dst, zeros_h).reshape(NC, NPAD, HID)
    z = _post_final(agg2, hws2, dinv, b2.reshape(1, HID))

    idx4 = jnp.concatenate([edge_index[0], edge_index[1],
                            neg_edge_index[0], neg_edge_index[1]])
    g = _lpgather_kernel(z, idx4).reshape(2, 2, E, HID)

    wa = lpW1[:HID]
    wb = lpW1[HID:]
    preds = _link_mlp(g, wa, wb, lpb1.reshape(1, 2 * HID), lpW2,
                      lpb2.reshape(1, HID), lpW3.reshape(1, HID),
                      lpb3.reshape(1, 1))
    return (preds[0], preds[1], z)


# speedup vs baseline: 4.8632x; 1.1276x over previous
import functools

from functools import partial
import jax
import jax.numpy as jnp
from jax import lax
from jax.experimental import pallas as pl
from jax.experimental.pallas import tpu as pltpu
from jax.experimental.pallas import tpu_sc as plsc

N = 10000
E = 320000
D_IN = 128
HID = 64

NC = 2
NS = 16
NW = NC * NS

CH = 80
EPT = E // NW
STEPS = EPT // CH
NPAD = 10240
RPT = NPAD // NS

DEGW = 16

def _dot(a, b):
    return jnp.dot(a.astype(jnp.bfloat16), b.astype(jnp.bfloat16),
                   preferred_element_type=jnp.float32)


_mesh = plsc.VectorSubcoreMesh(core_axis_name="c", subcore_axis_name="s")


def _wid():
    return lax.axis_index("c") * NS + lax.axis_index("s")


def _deg_body(dst_hbm, ones_hbm, zeros_hbm, out_hbm, idx_v, ones_v, acc_sp):
    sub = lax.axis_index("s")
    core = lax.axis_index("c")
    wid = core * NS + sub
    base = pl.multiple_of(wid * EPT, CH)

    r0 = pl.multiple_of(sub * RPT, RPT)
    pltpu.sync_copy(zeros_hbm.at[pl.ds(r0, RPT)], acc_sp.at[pl.ds(r0, RPT)])
    pltpu.sync_copy(ones_hbm, ones_v)
    plsc.subcore_barrier()

    def step(i, _):
        off = pl.multiple_of(base + i * CH, CH)
        pltpu.sync_copy(dst_hbm.at[pl.ds(off, CH)], idx_v)
        pltpu.sync_copy(ones_v, acc_sp.at[idx_v], add=True)
        return _

    lax.fori_loop(0, STEPS, step, None)
    plsc.subcore_barrier()
    pltpu.sync_copy(acc_sp.at[pl.ds(r0, RPT)],
                    out_hbm.at[pl.ds(core * NPAD + r0, RPT)])


_deg_kernel = pl.kernel(
    _deg_body,
    out_type=jax.ShapeDtypeStruct((NC * NPAD, DEGW), jnp.float32),
    mesh=_mesh,
    compiler_params=pltpu.CompilerParams(use_tc_tiling_on_sc=False),
    scratch_types=[
        pltpu.VMEM((CH,), jnp.int32),
        pltpu.VMEM((CH, DEGW), jnp.float32),
        pltpu.VMEM_SHARED((NPAD, DEGW), jnp.float32),
    ],
)


def _agg_body(hws_hbm, src_hbm, dst_hbm, zeros_hbm, out_hbm,
              sidx_v, didx_v, rows_v, acc_sp, sem):
    sub = lax.axis_index("s")
    core = lax.axis_index("c")
    wid = core * NS + sub
    base = pl.multiple_of(wid * EPT, CH)

    r0 = pl.multiple_of(sub * RPT, RPT)
    pltpu.sync_copy(zeros_hbm.at[pl.ds(r0, RPT)], acc_sp.at[pl.ds(r0, RPT)])
    plsc.subcore_barrier()

    def step(i, _):
        off = pl.multiple_of(base + i * CH, CH)
        pltpu.sync_copy(src_hbm.at[pl.ds(off, CH)], sidx_v)
        pltpu.sync_copy(dst_hbm.at[pl.ds(off, CH)], didx_v)
        pltpu.async_copy(hws_hbm.at[sidx_v], rows_v, sem).wait()
        pltpu.sync_copy(rows_v, acc_sp.at[didx_v], add=True)
        return _

    lax.fori_loop(0, STEPS, step, None)
    plsc.subcore_barrier()
    pltpu.sync_copy(acc_sp.at[pl.ds(r0, RPT)],
                    out_hbm.at[pl.ds(core * NPAD + r0, RPT)])


_agg_kernel = pl.kernel(
    _agg_body,
    out_type=jax.ShapeDtypeStruct((NC * NPAD, HID), jnp.float32),
    mesh=_mesh,
    compiler_params=pltpu.CompilerParams(use_tc_tiling_on_sc=False),
    scratch_types=[
        pltpu.VMEM((CH,), jnp.int32),
        pltpu.VMEM((CH,), jnp.int32),
        pltpu.VMEM((CH, HID), jnp.float32),
        pltpu.VMEM_SHARED((NPAD, HID), jnp.float32),
        pltpu.SemaphoreType.DMA,
    ],
)


NJOBS = 4


def _lpgather_body(z_hbm, idx_hbm, out_hbm, idx_v, rows_v, sem):
    wid = _wid()
    tbase = pl.multiple_of(wid * EPT, CH)

    def step(t, _):
        j = t // STEPS
        i = t - j * STEPS
        off = pl.multiple_of(j * E + tbase + i * CH, CH)
        pltpu.sync_copy(idx_hbm.at[pl.ds(off, CH)], idx_v)
        pltpu.async_copy(z_hbm.at[idx_v], rows_v, sem).wait()
        pltpu.sync_copy(rows_v, out_hbm.at[pl.ds(off, CH)])
        return _

    lax.fori_loop(0, NJOBS * STEPS, step, None)


_lpgather_kernel = pl.kernel(
    _lpgather_body,
    out_type=jax.ShapeDtypeStruct((NJOBS * E, HID), jnp.float32),
    mesh=_mesh,
    compiler_params=pltpu.CompilerParams(use_tc_tiling_on_sc=False),
    scratch_types=[
        pltpu.VMEM((CH,), jnp.int32),
        pltpu.VMEM((CH, HID), jnp.float32),
        pltpu.SemaphoreType.DMA,
    ],
)


def _prescale_body(degp_ref, x_ref, w_ref, hws_ref, dinv_ref):
    deg = degp_ref[0, 0:N, 0:1] + degp_ref[1, 0:N, 0:1] + 1.0
    dinv = 1.0 / jnp.sqrt(deg)
    hw = _dot(x_ref[...], w_ref[...])
    hws_ref[...] = hw * dinv
    dinv_ref[...] = dinv


def _prescale(degp, x, w):
    return pl.pallas_call(
        _prescale_body,
        out_shape=(jax.ShapeDtypeStruct((N, HID), jnp.float32),
                   jax.ShapeDtypeStruct((N, 1), jnp.float32)),
    )(degp, x, w)


def _post_mid_body(aggp_ref, hws_ref, dinv_ref, b_ref, w_ref, out_ref):
    acc = aggp_ref[0, 0:N] + aggp_ref[1, 0:N] + hws_ref[...]
    h = jnp.maximum(acc * dinv_ref[...] + b_ref[...], 0.0)
    out_ref[...] = _dot(h, w_ref[...]) * dinv_ref[...]


def _post_mid(aggp, hws, dinv, b, w):
    return pl.pallas_call(
        _post_mid_body,
        out_shape=jax.ShapeDtypeStruct((N, HID), jnp.float32),
    )(aggp, hws, dinv, b, w)


def _post_final_body(aggp_ref, hws_ref, dinv_ref, b_ref, z_ref):
    acc = aggp_ref[0, 0:N] + aggp_ref[1, 0:N] + hws_ref[...]
    z_ref[...] = jnp.maximum(acc * dinv_ref[...] + b_ref[...], 0.0)


def _post_final(aggp, hws, dinv, b):
    return pl.pallas_call(
        _post_final_body,
        out_shape=jax.ShapeDtypeStruct((N, HID), jnp.float32),
    )(aggp, hws, dinv, b)


EBLK = 2560
NEBLK = E // EBLK


def _mlp_body(g_ref, wa_ref, wb_ref, b1_ref, w2_ref, b2_ref, w3_ref, b3_ref,
              out_ref):
    for s in range(2):
        g0 = g_ref[s, 0]
        g1 = g_ref[s, 1]
        h1 = _dot(g0, wa_ref[...])
        h1 = h1 + _dot(g1, wb_ref[...])
        h1 = jnp.maximum(h1 + b1_ref[...], 0.0)
        h2 = _dot(h1, w2_ref[...])
        h2 = jnp.maximum(h2 + b2_ref[...], 0.0)
        h2b = h2.astype(jnp.bfloat16).astype(jnp.float32)
        w3b = w3_ref[...].astype(jnp.bfloat16).astype(jnp.float32)
        v = jnp.sum(h2b * w3b, axis=1) + b3_ref[0, 0]
        out_ref[s, :] = v


def _link_mlp(g, wa, wb, b1, w2, b2, w3, b3):
    return pl.pallas_call(
        _mlp_body,
        grid=(NEBLK,),
        in_specs=[
            pl.BlockSpec((2, 2, EBLK, HID), lambda i: (0, 0, i, 0)),
            pl.BlockSpec((HID, 2 * HID), lambda i: (0, 0)),
            pl.BlockSpec((HID, 2 * HID), lambda i: (0, 0)),
            pl.BlockSpec((1, 2 * HID), lambda i: (0, 0)),
            pl.BlockSpec((2 * HID, HID), lambda i: (0, 0)),
            pl.BlockSpec((1, HID), lambda i: (0, 0)),
            pl.BlockSpec((1, HID), lambda i: (0, 0)),
            pl.BlockSpec((1, 1), lambda i: (0, 0)),
        ],
        out_specs=pl.BlockSpec((2, EBLK), lambda i: (0, i)),
        out_shape=jax.ShapeDtypeStruct((2, E), jnp.float32),
    )(g, wa, wb, b1, w2, b2, w3, b3)


def kernel(x, edge_index, neg_edge_index, W1, b1, W2, b2,
           lpW1, lpb1, lpW2, lpb2, lpW3, lpb3):
    src = edge_index[0]
    dst = edge_index[1]
    zeros_h = jnp.zeros((NPAD, HID), jnp.float32)
    zeros_d = jnp.zeros((NPAD, DEGW), jnp.float32)
    ones_d = jnp.ones((CH, DEGW), jnp.float32)

    degp = _deg_kernel(dst, ones_d, zeros_d)
    degp = degp.reshape(NC, NPAD, DEGW)

    hws1, dinv = _prescale(degp, x, W1)
    agg1 = _agg_kernel(hws1, src, dst, zeros_h).reshape(NC, NPAD, HID)
    hws2 = _post_mid(agg1, hws1, dinv, b1.reshape(1, HID), W2)
    agg2 = _agg_kernel(hws2, src, dst, zeros_h).reshape(NC, NPAD, HID)
    z = _post_final(agg2, hws2, dinv, b2.reshape(1, HID))

    idx4 = jnp.concatenate([edge_index[0], edge_index[1],
                            neg_edge_index[0], neg_edge_index[1]])
    g = _lpgather_kernel(z, idx4).reshape(2, 2, E, HID)

    wa = lpW1[:HID]
    wb = lpW1[HID:]
    preds = _link_mlp(g, wa, wb, lpb1.reshape(1, 2 * HID), lpW2,
                      lpb2.reshape(1, HID), lpW3.reshape(1, HID),
                      lpb3.reshape(1, 1))
    return (preds[0], preds[1], z)


# speedup vs baseline: 6.4537x; 1.3270x over previous
import functools

import jax
import jax.numpy as jnp
from jax import lax
from jax.experimental import pallas as pl
from jax.experimental.pallas import tpu as pltpu
from jax.experimental.pallas import tpu_sc as plsc

N = 10000
E = 320000
D_IN = 128
HID = 64

NC = 2
NS = 16
NW = NC * NS

CH = 80
EPT = E // NW
STEPS = EPT // CH
K = 5
GROUPS = STEPS // K
NPAD = 10240
RPT = NPAD // NS

DEGW = 16

NJOBS = 4


def _dot(a, b):
    return jnp.dot(a.astype(jnp.bfloat16), b.astype(jnp.bfloat16),
                   preferred_element_type=jnp.float32)


_mesh = plsc.VectorSubcoreMesh(core_axis_name="c", subcore_axis_name="s")


def _deg_body(dst_hbm, ones_hbm, zeros_hbm, out_hbm,
              idx_all, ones_v, acc_sp, ssem):
    sub = lax.axis_index("s")
    core = lax.axis_index("c")
    wid = core * NS + sub

    r0 = pl.multiple_of(sub * RPT, RPT)
    pltpu.sync_copy(zeros_hbm.at[pl.ds(r0, RPT)], acc_sp.at[pl.ds(r0, RPT)])
    pltpu.sync_copy(dst_hbm.at[wid], idx_all)
    pltpu.sync_copy(ones_hbm, ones_v)
    plsc.subcore_barrier()

    def grp(g, _):
        base = g * K
        descs = [
            pltpu.async_copy(ones_v, acc_sp.at[idx_all.at[base + b]],
                             ssem, add=True)
            for b in range(K)
        ]
        for d in descs:
            d.wait()
        return _

    lax.fori_loop(0, GROUPS, grp, None)
    plsc.subcore_barrier()
    pltpu.sync_copy(acc_sp.at[pl.ds(r0, RPT)],
                    out_hbm.at[pl.ds(core * NPAD + r0, RPT)])


_deg_kernel = pl.kernel(
    _deg_body,
    out_type=jax.ShapeDtypeStruct((NC * NPAD, DEGW), jnp.float32),
    mesh=_mesh,
    compiler_params=pltpu.CompilerParams(use_tc_tiling_on_sc=False),
    scratch_types=[
        pltpu.VMEM((STEPS, CH), jnp.int32),
        pltpu.VMEM((CH, DEGW), jnp.float32),
        pltpu.VMEM_SHARED((NPAD, DEGW), jnp.float32),
        pltpu.SemaphoreType.DMA,
    ],
)


def _agg_body(hws_hbm, src_hbm, dst_hbm, zeros_hbm, out_hbm,
              sidx_all, didx_all, rows_v, acc_sp, gsem, ssem):
    sub = lax.axis_index("s")
    core = lax.axis_index("c")
    wid = core * NS + sub

    r0 = pl.multiple_of(sub * RPT, RPT)
    pltpu.sync_copy(zeros_hbm.at[pl.ds(r0, RPT)], acc_sp.at[pl.ds(r0, RPT)])
    pltpu.sync_copy(src_hbm.at[wid], sidx_all)
    pltpu.sync_copy(dst_hbm.at[wid], didx_all)
    plsc.subcore_barrier()

    def grp(g, _):
        base = g * K
        gds = [
            pltpu.async_copy(hws_hbm.at[sidx_all.at[base + b]],
                             rows_v.at[b], gsem)
            for b in range(K)
        ]
        for d in gds:
            d.wait()
        sds = [
            pltpu.async_copy(rows_v.at[b], acc_sp.at[didx_all.at[base + b]],
                             ssem, add=True)
            for b in range(K)
        ]
        for d in sds:
            d.wait()
        return _

    lax.fori_loop(0, GROUPS, grp, None)
    plsc.subcore_barrier()
    pltpu.sync_copy(acc_sp.at[pl.ds(r0, RPT)],
                    out_hbm.at[pl.ds(core * NPAD + r0, RPT)])


_agg_kernel = pl.kernel(
    _agg_body,
    out_type=jax.ShapeDtypeStruct((NC * NPAD, HID), jnp.float32),
    mesh=_mesh,
    compiler_params=pltpu.CompilerParams(use_tc_tiling_on_sc=False),
    scratch_types=[
        pltpu.VMEM((STEPS, CH), jnp.int32),
        pltpu.VMEM((STEPS, CH), jnp.int32),
        pltpu.VMEM((K, CH, HID), jnp.float32),
        pltpu.VMEM_SHARED((NPAD, HID), jnp.float32),
        pltpu.SemaphoreType.DMA,
        pltpu.SemaphoreType.DMA,
    ],
)


def _lpgather_body(z_hbm, idx_hbm, out_hbm, idx_all, rows_v, gsem, ssem):
    sub = lax.axis_index("s")
    core = lax.axis_index("c")
    wid = core * NS + sub
    tbase = pl.multiple_of(wid * EPT, CH)

    for j in range(NJOBS):
        pltpu.sync_copy(idx_hbm.at[j, wid], idx_all.at[j])

    def run_job(j):
        def grp(g, _):
            base = g * K
            gds = [
                pltpu.async_copy(z_hbm.at[idx_all.at[j, base + b]],
                                 rows_v.at[b], gsem)
                for b in range(K)
            ]
            for d in gds:
                d.wait()
            sds = [
                pltpu.async_copy(
                    rows_v.at[b],
                    out_hbm.at[j, pl.ds(tbase + (base + b) * CH, CH)],
                    ssem)
                for b in range(K)
            ]
            for d in sds:
                d.wait()
            return _

        lax.fori_loop(0, GROUPS, grp, None)

    for j in range(NJOBS):
        run_job(j)


_lpgather_kernel = pl.kernel(
    _lpgather_body,
    out_type=jax.ShapeDtypeStruct((NJOBS, E, HID), jnp.bfloat16),
    mesh=_mesh,
    compiler_params=pltpu.CompilerParams(use_tc_tiling_on_sc=False),
    scratch_types=[
        pltpu.VMEM((NJOBS, STEPS, CH), jnp.int32),
        pltpu.VMEM((K, CH, HID), jnp.bfloat16),
        pltpu.SemaphoreType.DMA,
        pltpu.SemaphoreType.DMA,
    ],
)


def _prescale_body(degp_ref, x_ref, w_ref, hws_ref, dinv_ref):
    deg = degp_ref[0, 0:N, 0:1] + degp_ref[1, 0:N, 0:1] + 1.0
    dinv = 1.0 / jnp.sqrt(deg)
    hw = _dot(x_ref[...], w_ref[...])
    hws_ref[...] = hw * dinv
    dinv_ref[...] = dinv


def _prescale(degp, x, w):
    return pl.pallas_call(
        _prescale_body,
        out_shape=(jax.ShapeDtypeStruct((N, HID), jnp.float32),
                   jax.ShapeDtypeStruct((N, 1), jnp.float32)),
    )(degp, x, w)


def _post_mid_body(aggp_ref, hws_ref, dinv_ref, b_ref, w_ref, out_ref):
    acc = aggp_ref[0, 0:N] + aggp_ref[1, 0:N] + hws_ref[...]
    h = jnp.maximum(acc * dinv_ref[...] + b_ref[...], 0.0)
    out_ref[...] = _dot(h, w_ref[...]) * dinv_ref[...]


def _post_mid(aggp, hws, dinv, b, w):
    return pl.pallas_call(
        _post_mid_body,
        out_shape=jax.ShapeDtypeStruct((N, HID), jnp.float32),
    )(aggp, hws, dinv, b, w)


def _post_final_body(aggp_ref, hws_ref, dinv_ref, b_ref, z_ref, zb_ref):
    acc = aggp_ref[0, 0:N] + aggp_ref[1, 0:N] + hws_ref[...]
    z = jnp.maximum(acc * dinv_ref[...] + b_ref[...], 0.0)
    z_ref[...] = z
    zb_ref[...] = z.astype(jnp.bfloat16)


def _post_final(aggp, hws, dinv, b):
    return pl.pallas_call(
        _post_final_body,
        out_shape=(jax.ShapeDtypeStruct((N, HID), jnp.float32),
                   jax.ShapeDtypeStruct((N, HID), jnp.bfloat16)),
    )(aggp, hws, dinv, b)


EBLK = 2560
NEBLK = E // EBLK


def _mlp_body(g_ref, wa_ref, wb_ref, b1_ref, w2_ref, b2_ref, w3_ref, b3_ref,
              out_ref):
    wa = wa_ref[...].astype(jnp.bfloat16)
    wb = wb_ref[...].astype(jnp.bfloat16)
    w2 = w2_ref[...].astype(jnp.bfloat16)
    w3b = w3_ref[...].astype(jnp.bfloat16).astype(jnp.float32)
    for s in range(2):
        g0 = g_ref[2 * s]
        g1 = g_ref[2 * s + 1]
        h1 = jnp.dot(g0, wa, preferred_element_type=jnp.float32)
        h1 = h1 + jnp.dot(g1, wb, preferred_element_type=jnp.float32)
        h1 = jnp.maximum(h1 + b1_ref[...], 0.0)
        h2 = jnp.dot(h1.astype(jnp.bfloat16), w2,
                     preferred_element_type=jnp.float32)
        h2 = jnp.maximum(h2 + b2_ref[...], 0.0)
        h2b = h2.astype(jnp.bfloat16).astype(jnp.float32)
        v = jnp.sum(h2b * w3b, axis=1) + b3_ref[0, 0]
        out_ref[s, :] = v


def _link_mlp(g, wa, wb, b1, w2, b2, w3, b3):
    return pl.pallas_call(
        _mlp_body,
        grid=(NEBLK,),
        in_specs=[
            pl.BlockSpec((NJOBS, EBLK, HID), lambda i: (0, i, 0)),
            pl.BlockSpec((HID, 2 * HID), lambda i: (0, 0)),
            pl.BlockSpec((HID, 2 * HID), lambda i: (0, 0)),
            pl.BlockSpec((1, 2 * HID), lambda i: (0, 0)),
            pl.BlockSpec((2 * HID, HID), lambda i: (0, 0)),
            pl.BlockSpec((1, HID), lambda i: (0, 0)),
            pl.BlockSpec((1, HID), lambda i: (0, 0)),
            pl.BlockSpec((1, 1), lambda i: (0, 0)),
        ],
        out_specs=pl.BlockSpec((2, EBLK), lambda i: (0, i)),
        out_shape=jax.ShapeDtypeStruct((2, E), jnp.float32),
    )(g, wa, wb, b1, w2, b2, w3, b3)


def kernel(x, edge_index, neg_edge_index, W1, b1, W2, b2,
           lpW1, lpb1, lpW2, lpb2, lpW3, lpb3):
    src3 = edge_index[0].reshape(NW, STEPS, CH)
    dst3 = edge_index[1].reshape(NW, STEPS, CH)
    zeros_h = jnp.zeros((NPAD, HID), jnp.float32)
    zeros_d = jnp.zeros((NPAD, DEGW), jnp.float32)
    ones_d = jnp.ones((CH, DEGW), jnp.float32)

    degp = _deg_kernel(dst3, ones_d, zeros_d)
    degp = degp.reshape(NC, NPAD, DEGW)

    hws1, dinv = _prescale(degp, x, W1)
    agg1 = _agg_kernel(hws1, src3, dst3, zeros_h).reshape(NC, NPAD, HID)
    hws2 = _post_mid(agg1, hws1, dinv, b1.reshape(1, HID), W2)
    agg2 = _agg_kernel(hws2, src3, dst3, zeros_h).reshape(NC, NPAD, HID)
    z, z_bf = _post_final(agg2, hws2, dinv, b2.reshape(1, HID))

    idx4 = jnp.stack([edge_index[0], edge_index[1],
                      neg_edge_index[0], neg_edge_index[1]])
    idx4 = idx4.reshape(NJOBS, NW, STEPS, CH)
    g = _lpgather_kernel(z_bf, idx4)

    wa = lpW1[:HID]
    wb = lpW1[HID:]
    preds = _link_mlp(g, wa, wb, lpb1.reshape(1, 2 * HID), lpW2,
                      lpb2.reshape(1, HID), lpW3.reshape(1, HID),
                      lpb3.reshape(1, 1))
    return (preds[0], preds[1], z)
